# baseline (speedup 1.0000x reference)
import jax
import jax.numpy as jnp
from jax import lax
from jax.experimental import pallas as pl
from jax.experimental.pallas import tpu as pltpu
from jax.experimental.pallas import tpu_sc as plsc

_B, _N, _D = 128, 500, 256
_NC, _NS = 2, 16
_NW = _NC * _NS
_T = 128
_S = _N - _T
_RPW = _T // _NW
_BB = 16


def _sc_fill_body(table_hbm, idx_hbm, buf_hbm, idx_v, rep_v, gsem, wsem):
    wid = lax.axis_index("s") * _NC + lax.axis_index("c")
    writes = [None, None]
    for k in range(_RPW):
        slot = k % 2
        if k >= 2:
            writes[slot].wait()
        pltpu.sync_copy(
            idx_hbm.at[pl.ds(wid * (_RPW * _B) + k * _B, _B)], idx_v
        )
        pltpu.async_copy(table_hbm.at[idx_v], rep_v.at[slot], gsem).wait()
        writes[slot] = pltpu.async_copy(
            rep_v.at[slot], buf_hbm.at[_S + wid * _RPW + k], wsem
        )
    writes[0].wait()
    writes[1].wait()


def _tc_body(emb_ref, out_ref):
    out_ref[...] = jnp.broadcast_to(
        emb_ref[pl.ds(0, _S), :][:, None, :], (_S, _BB, _D)
    )


@jax.jit
def _bcast(embed_weight):
    idx = jnp.repeat(jnp.arange(_S, _N, dtype=jnp.int32), _B)

    tc_part = pl.pallas_call(
        _tc_body,
        grid=(_B // _BB,),
        in_specs=[pl.BlockSpec((_N, _D), lambda j: (0, 0))],
        out_specs=pl.BlockSpec((_S, _BB, _D), lambda j: (0, j, 0)),
        out_shape=jax.ShapeDtypeStruct((_N, _B, _D), jnp.float32),
    )(embed_weight)

    buf = jax.new_ref(tc_part)
    mesh = plsc.VectorSubcoreMesh(core_axis_name="c", subcore_axis_name="s")
    pl.kernel(
        _sc_fill_body,
        out_type=(),
        mesh=mesh,
        scratch_types=[
            pltpu.VMEM((_B,), jnp.int32),
            pltpu.VMEM((2, _B, _D), jnp.float32),
            pltpu.SemaphoreType.DMA,
            pltpu.SemaphoreType.DMA,
        ],
    )(embed_weight, idx, buf)

    rows_major = buf[...]
    return jnp.transpose(rows_major, (1, 0, 2))


def kernel(x, embed_weight):
    del x
    return _bcast(embed_weight)


# speedup vs baseline: 1.9544x; 1.9544x over previous
import jax
import jax.numpy as jnp
from jax import lax
from jax.experimental import pallas as pl
from jax.experimental.pallas import tpu as pltpu
from jax.experimental.pallas import tpu_sc as plsc

_B, _N, _D = 128, 500, 256
_NC, _NS = 2, 16
_NW = _NC * _NS
_T = 128
_S = _N - _T
_RPW = _T // _NW
_REP = 16
_BB = 16


def _sc_fill_body(table_hbm, buf_hbm, rep_v, rsem, wsem):
    wid = lax.axis_index("s") * _NC + lax.axis_index("c")
    r0 = _S + wid * _RPW
    span = pl.multiple_of((r0 // 8) * 8, 8)
    off = r0 - span

    @pl.when(wid < _NW - 1)
    def _stage_main():
        reads = [
            pltpu.async_copy(
                table_hbm.at[pl.ds(span, 8)], rep_v.at[:, j, :], rsem
            )
            for j in range(_REP)
        ]
        for r in reads:
            r.wait()

    @pl.when(wid == _NW - 1)
    def _stage_tail():
        reads = [
            pltpu.async_copy(
                table_hbm.at[pl.ds(_N - _RPW, _RPW)],
                rep_v.at[pl.ds(0, _RPW), j, :],
                rsem,
            )
            for j in range(_REP)
        ]
        for r in reads:
            r.wait()

    writes = [
        pltpu.async_copy(
            rep_v.at[pl.ds(off, _RPW)],
            buf_hbm.at[pl.ds(r0, _RPW), pl.ds(j * _REP, _REP), :],
            wsem,
        )
        for j in range(_B // _REP)
    ]
    for w in writes:
        w.wait()


def _tc_body(emb_ref, out_ref):
    out_ref[...] = jnp.broadcast_to(
        emb_ref[pl.ds(0, _S), :][:, None, :], (_S, _BB, _D)
    )


@jax.jit
def _bcast(embed_weight):
    tc_part = pl.pallas_call(
        _tc_body,
        grid=(_B // _BB,),
        in_specs=[pl.BlockSpec((_N, _D), lambda j: (0, 0))],
        out_specs=pl.BlockSpec((_S, _BB, _D), lambda j: (0, j, 0)),
        out_shape=jax.ShapeDtypeStruct((_N, _B, _D), jnp.float32),
    )(embed_weight)

    buf = jax.new_ref(tc_part)
    mesh = plsc.VectorSubcoreMesh(core_axis_name="c", subcore_axis_name="s")
    pl.kernel(
        _sc_fill_body,
        out_type=(),
        mesh=mesh,
        scratch_types=[
            pltpu.VMEM((8, _REP, _D), jnp.float32),
            pltpu.SemaphoreType.DMA,
            pltpu.SemaphoreType.DMA,
        ],
    )(embed_weight, buf)

    rows_major = buf[...]
    return jnp.transpose(rows_major, (1, 0, 2))


def kernel(x, embed_weight):
    del x
    return _bcast(embed_weight)


# speedup vs baseline: 2.1919x; 1.1215x over previous
import jax
import jax.numpy as jnp
from jax import lax
from jax.experimental import pallas as pl
from jax.experimental.pallas import tpu as pltpu
from jax.experimental.pallas import tpu_sc as plsc

_B, _N, _D = 128, 500, 256
_NC, _NS = 2, 16
_T = 128
_GW = 16
_GR = _T // _GW
_RB = 128
_BB = 16


def _sc_gather_body(table_hbm, idx_hbm, out_hbm, idx_v, rows_v, sem):
    wid = lax.axis_index("s") * _NC + lax.axis_index("c")

    @pl.when(wid < _GW)
    def _():
        base = wid * _GR
        pltpu.sync_copy(idx_hbm.at[pl.ds(base, _GR)], idx_v)
        pltpu.async_copy(table_hbm.at[idx_v], rows_v, sem).wait()
        pltpu.sync_copy(rows_v, out_hbm.at[pl.ds(base, _GR)])


def _tc_body(emb_ref, out_ref):
    out_ref[...] = jnp.broadcast_to(emb_ref[...][:, None, :], (_RB, _BB, _D))


def _tc2_body(full_ref, g_ref, out_ref):
    del full_ref
    out_ref[...] = jnp.broadcast_to(g_ref[...][:, None, :], (_RB, _BB, _D))


@jax.jit
def _bcast(embed_weight):
    idx = jnp.arange(_T, dtype=jnp.int32)
    mesh = plsc.VectorSubcoreMesh(core_axis_name="c", subcore_axis_name="s")
    gathered = pl.kernel(
        _sc_gather_body,
        mesh=mesh,
        out_type=jax.ShapeDtypeStruct((_T, _D), jnp.float32),
        scratch_types=[
            pltpu.VMEM((_GR,), jnp.int32),
            pltpu.VMEM((_GR, _D), jnp.float32),
            pltpu.SemaphoreType.DMA,
        ],
    )(embed_weight, idx)

    tc1 = pl.pallas_call(
        _tc_body,
        grid=(3, _B // _BB),
        in_specs=[pl.BlockSpec((_RB, _D), lambda i, j: (1 + i, 0))],
        out_specs=pl.BlockSpec((_RB, _BB, _D), lambda i, j: (1 + i, j, 0)),
        out_shape=jax.ShapeDtypeStruct((_N, _B, _D), jnp.float32),
    )(embed_weight)

    rows_major = pl.pallas_call(
        _tc2_body,
        grid=(_B // _BB,),
        in_specs=[
            pl.BlockSpec(memory_space=pl.ANY),
            pl.BlockSpec((_T, _D), lambda j: (0, 0)),
        ],
        out_specs=pl.BlockSpec((_RB, _BB, _D), lambda j: (0, j, 0)),
        out_shape=jax.ShapeDtypeStruct((_N, _B, _D), jnp.float32),
        input_output_aliases={0: 0},
    )(tc1, gathered)

    return jnp.transpose(rows_major, (1, 0, 2))


def kernel(x, embed_weight):
    del x
    return _bcast(embed_weight)


# speedup vs baseline: 2.3880x; 1.0895x over previous
import jax
import jax.numpy as jnp
from jax import lax
from jax.experimental import pallas as pl
from jax.experimental.pallas import tpu as pltpu
from jax.experimental.pallas import tpu_sc as plsc

_B, _N, _D = 128, 500, 256
_NC, _NS = 2, 16
_S = 375
_T = _N - _S
_GP = 128
_GW = 16
_GR = _GP // _GW
_BB1 = 32
_BB2 = 32


def _sc_gather_body(table_hbm, idx_hbm, out_hbm, idx_v, rows_v, sem):
    wid = lax.axis_index("s") * _NC + lax.axis_index("c")

    @pl.when(wid < _GW)
    def _():
        base = wid * _GR
        pltpu.sync_copy(idx_hbm.at[pl.ds(base, _GR)], idx_v)
        pltpu.async_copy(table_hbm.at[idx_v], rows_v, sem).wait()
        pltpu.sync_copy(rows_v, out_hbm.at[pl.ds(base, _GR)])


def _tc1_body(emb_ref, out_ref):
    out_ref[...] = jnp.broadcast_to(
        emb_ref[pl.ds(0, _S), :][:, None, :], (_S, _BB1, _D)
    )


def _tc2_body(full_ref, g_ref, out_ref):
    del full_ref
    out_ref[...] = jnp.broadcast_to(
        g_ref[pl.ds(0, _T), :][:, None, :], (_T, _BB2, _D)
    )


@jax.jit
def _bcast(embed_weight):
    idx = jnp.minimum(_S + jnp.arange(_GP, dtype=jnp.int32), _N - 1)
    mesh = plsc.VectorSubcoreMesh(core_axis_name="c", subcore_axis_name="s")
    gathered = pl.kernel(
        _sc_gather_body,
        mesh=mesh,
        out_type=jax.ShapeDtypeStruct((_GP, _D), jnp.float32),
        scratch_types=[
            pltpu.VMEM((_GR,), jnp.int32),
            pltpu.VMEM((_GR, _D), jnp.float32),
            pltpu.SemaphoreType.DMA,
        ],
    )(embed_weight, idx)

    tc1 = pl.pallas_call(
        _tc1_body,
        grid=(_B // _BB1,),
        in_specs=[pl.BlockSpec((_N, _D), lambda j: (0, 0))],
        out_specs=pl.BlockSpec((_S, _BB1, _D), lambda j: (0, j, 0)),
        out_shape=jax.ShapeDtypeStruct((_N, _B, _D), jnp.float32),
    )(embed_weight)

    rows_major = pl.pallas_call(
        _tc2_body,
        grid=(_B // _BB2,),
        in_specs=[
            pl.BlockSpec(memory_space=pl.ANY),
            pl.BlockSpec((_GP, _D), lambda j: (0, 0)),
        ],
        out_specs=pl.BlockSpec((_T, _BB2, _D), lambda j: (_S // _T, j, 0)),
        out_shape=jax.ShapeDtypeStruct((_N, _B, _D), jnp.float32),
        input_output_aliases={0: 0},
    )(tc1, gathered)

    return jnp.transpose(rows_major, (1, 0, 2))


def kernel(x, embed_weight):
    del x
    return _bcast(embed_weight)
